# baseline (speedup 1.0000x reference)
import functools

import jax
import jax.numpy as jnp
from jax import lax
from jax.experimental import pallas as pl
from jax.experimental.pallas import tpu as pltpu
from jax.experimental.pallas import tpu_sc as plsc

VOCAB = 1000000
EMB = 64
MAX_LEN = 200
CLASSES = 100
BATCH = 4096

NC = 2
NS = 16
NW = NC * NS
B_PER_W = BATCH // NW
TOK_PER_W = B_PER_W * MAX_LEN
S0 = 128
S1 = MAX_LEN - S0

FC = 256
BULK = VOCAB - VOCAB % 128
NBLK = BULK // FC
TAIL = VOCAB - BULK
BLK_PER_W = (NBLK + NW - 1) // NW


def _fmt_body(tab_t_hbm, tail_hbm, out_hbm, in0, in1, out0, out1,
              tail_in, tail_out, sem0, sem1, wsem0, wsem1):
    wid = lax.axis_index("s") * NC + lax.axis_index("c")
    iota = lax.iota(jnp.int32, 16)

    rowoff = [(16 * k + iota) * FC for k in range(4)]

    def transpose_block(in_v, out_v):
        @plsc.parallel_loop(0, FC, step=1, unroll=8)
        def col_body(r):
            cv = jnp.full((16,), r, jnp.int32)
            for k in range(4):
                vals = plsc.load_gather(in_v, [16 * k + iota, cv])
                out_v[r, pl.ds(16 * k, 16)] = vals

    def fire(b, in_v, sem):
        pltpu.async_copy(tab_t_hbm.at[:, pl.ds(FC * b, FC)],
                         in_v.at[:, pl.ds(0, FC)], sem)

    def drain(in_v, sem):
        pltpu.make_async_copy(tab_t_hbm.at[:, pl.ds(0, FC)],
                              in_v.at[:, pl.ds(0, FC)], sem).wait()

    def put(b, out_v, sem):
        pltpu.async_copy(out_v, out_hbm.at[pl.ds(FC * b, FC)], sem)

    def put_drain(out_v, sem):
        pltpu.make_async_copy(out_v, out_hbm.at[pl.ds(0, FC)], sem).wait()

    fire(wid, in0, sem0)

    def pair_body(t, _):
        b0 = wid + NW * 2 * t

        @pl.when(b0 + NW < NBLK)
        def _():
            fire(b0 + NW, in1, sem1)

        @pl.when(b0 < NBLK)
        def _():
            drain(in0, sem0)

            @pl.when(t > 0)
            def _():
                put_drain(out0, wsem0)
            transpose_block(in0, out0)
            put(b0, out0, wsem0)

        @pl.when(b0 + 2 * NW < NBLK)
        def _():
            fire(b0 + 2 * NW, in0, sem0)

        @pl.when(b0 + NW < NBLK)
        def _():
            drain(in1, sem1)

            @pl.when(t > 0)
            def _():
                put_drain(out1, wsem1)
            transpose_block(in1, out1)
            put(b0 + NW, out1, wsem1)
        return 0

    lax.fori_loop(0, (BLK_PER_W + 1) // 2, pair_body, 0)
    put_drain(out0, wsem0)
    put_drain(out1, wsem1)

    @pl.when(wid == 0)
    def _():
        pltpu.sync_copy(tail_hbm, tail_in)

        def tail_body(r, _):
            for k in range(4):
                tail_out[r, pl.ds(16 * k, 16)] = tail_in[r, pl.ds(16 * k, 16)]
            return 0
        lax.fori_loop(0, TAIL, tail_body, 0)
        pltpu.sync_copy(tail_out, out_hbm.at[pl.ds(BULK, TAIL)])


_fmt_call = functools.partial(
    pl.kernel,
    out_type=jax.ShapeDtypeStruct((VOCAB, 2 * EMB), jnp.float32),
    mesh=plsc.VectorSubcoreMesh(core_axis_name="c", subcore_axis_name="s"),
    compiler_params=pltpu.CompilerParams(needs_layout_passes=False,
                                         disable_bounds_checks=True),
    scratch_types=[
        pltpu.VMEM((EMB, FC + 1), jnp.float32),
        pltpu.VMEM((EMB, FC + 1), jnp.float32),
        pltpu.VMEM((FC, 2 * EMB), jnp.float32),
        pltpu.VMEM((FC, 2 * EMB), jnp.float32),
        pltpu.VMEM((TAIL, EMB), jnp.float32),
        pltpu.VMEM((TAIL, 2 * EMB), jnp.float32),
        pltpu.SemaphoreType.DMA,
        pltpu.SemaphoreType.DMA,
        pltpu.SemaphoreType.DMA,
        pltpu.SemaphoreType.DMA,
    ],
)(_fmt_body)


def _pool_body(idx_hbm, table_hbm, out_hbm, idx_v, buf0, buf1, stage,
               sem0, sem1):
    wid = lax.axis_index("s") * NC + lax.axis_index("c")
    base = wid * B_PER_W

    pltpu.sync_copy(idx_hbm.at[pl.ds(wid * TOK_PER_W, TOK_PER_W)], idx_v)

    def fire(i, buf, sem):
        tok = i * MAX_LEN
        pltpu.async_copy(table_hbm.at[idx_v.at[pl.ds(tok, S0)]],
                         buf.at[pl.ds(0, S0)], sem)
        pltpu.async_copy(table_hbm.at[idx_v.at[pl.ds(tok + S0, S1)]],
                         buf.at[pl.ds(S0, S1)], sem)

    def drain(buf, sem):
        pltpu.make_async_copy(table_hbm.at[idx_v.at[pl.ds(0, S0)]],
                              buf.at[pl.ds(0, S0)], sem).wait()
        pltpu.make_async_copy(table_hbm.at[idx_v.at[pl.ds(0, S1)]],
                              buf.at[pl.ds(S0, S1)], sem).wait()

    zero = jnp.zeros((16,), jnp.float32)

    def reduce_item(i, buf):
        def red(m, accs):
            a = tuple(
                accs[k] + buf[2 * m, pl.ds(16 * k, 16)] for k in range(4)
            )
            b = tuple(
                accs[4 + k] + buf[2 * m + 1, pl.ds(16 * k, 16)]
                for k in range(4)
            )
            return a + b
        accs = lax.fori_loop(0, MAX_LEN // 2, red, (zero,) * 8)
        for k in range(4):
            stage[i, pl.ds(16 * k, 16)] = accs[k] + accs[4 + k]

    fire(0, buf0, sem0)

    def pair_body(g, _):
        i0 = 2 * g
        fire(i0 + 1, buf1, sem1)
        drain(buf0, sem0)
        reduce_item(i0, buf0)

        @pl.when(g < B_PER_W // 2 - 1)
        def _():
            fire(i0 + 2, buf0, sem0)
        drain(buf1, sem1)
        reduce_item(i0 + 1, buf1)
        return 0

    lax.fori_loop(0, B_PER_W // 2, pair_body, 0)
    pltpu.sync_copy(stage, out_hbm.at[pl.ds(base, B_PER_W)])


_pool_call = functools.partial(
    pl.kernel,
    out_type=jax.ShapeDtypeStruct((BATCH, EMB), jnp.float32),
    mesh=plsc.VectorSubcoreMesh(core_axis_name="c", subcore_axis_name="s"),
    scratch_types=[
        pltpu.VMEM((TOK_PER_W,), jnp.int32),
        pltpu.VMEM((MAX_LEN, 2 * EMB), jnp.float32),
        pltpu.VMEM((MAX_LEN, 2 * EMB), jnp.float32),
        pltpu.VMEM((B_PER_W, EMB), jnp.float32),
        pltpu.SemaphoreType.DMA,
        pltpu.SemaphoreType.DMA,
    ],
)(_pool_body)


CPAD = 128
_DBLK = 512


def _dense_kernel(x_ref, w_ref, b_ref, o_ref):
    logits = jnp.dot(x_ref[...], w_ref[...],
                     preferred_element_type=jnp.float32) + b_ref[...]
    m = jnp.max(logits, axis=-1, keepdims=True)
    e = jnp.exp(logits - m)
    o_ref[...] = e / jnp.sum(e, axis=-1, keepdims=True)


_dense_call = pl.pallas_call(
    _dense_kernel,
    grid=(BATCH // _DBLK,),
    in_specs=[
        pl.BlockSpec((_DBLK, EMB), lambda i: (i, 0)),
        pl.BlockSpec((EMB, CPAD), lambda i: (0, 0)),
        pl.BlockSpec((1, CPAD), lambda i: (0, 0)),
    ],
    out_specs=pl.BlockSpec((_DBLK, CPAD), lambda i: (i, 0)),
    out_shape=jax.ShapeDtypeStruct((BATCH, CPAD), jnp.float32),
)


def kernel(inputs, table, W, b):
    idx = inputs.astype(jnp.int32).reshape(-1)
    table_t = table.T
    tail = table[BULK:, :]
    table_wide = _fmt_call(table_t, tail)
    pool_sum = _pool_call(idx, table_wide)
    w_pad = jnp.pad(W * (1.0 / MAX_LEN), ((0, 0), (0, CPAD - CLASSES)))
    b_pad = jnp.concatenate(
        [b, jnp.full((CPAD - CLASSES,), -1e30, b.dtype)]).reshape(1, CPAD)
    out = _dense_call(pool_sum, w_pad, b_pad)
    return out[:, :CLASSES]


# speedup vs baseline: 1.3836x; 1.3836x over previous
import functools

import jax
import jax.numpy as jnp
from jax import lax
from jax.experimental import pallas as pl
from jax.experimental.pallas import tpu as pltpu
from jax.experimental.pallas import tpu_sc as plsc

VOCAB = 1000000
EMB = 64
MAX_LEN = 200
CLASSES = 100
BATCH = 4096

NC = 2
NS = 16
NW = NC * NS
B_PER_W = BATCH // NW
TOK_PER_W = B_PER_W * MAX_LEN
S0 = 128
S1 = MAX_LEN - S0

FC = 256
BULK = VOCAB - VOCAB % 128
NBLK = BULK // FC
TAIL = VOCAB - BULK
BLK_PER_W = (NBLK + NW - 1) // NW


def _fmt_body(tab_t_hbm, tail_hbm, out_hbm, in0, in1, out0, out1,
              tail_in, tail_out, sem0, sem1, wsem0, wsem1):
    wid = lax.axis_index("s") * NC + lax.axis_index("c")
    iota = lax.iota(jnp.int32, 16)

    rowoff = [(16 * k + iota) * FC for k in range(4)]

    def transpose_block(in_v, out_v):
        @plsc.parallel_loop(0, FC, step=1, unroll=8)
        def col_body(r):
            cv = jnp.full((16,), r, jnp.int32)
            for k in range(4):
                vals = plsc.load_gather(in_v, [16 * k + iota, cv])
                out_v[r, pl.ds(16 * k, 16)] = vals

    def fire(b, in_v, sem):
        pltpu.async_copy(tab_t_hbm.at[:, pl.ds(FC * b, FC)],
                         in_v.at[:, pl.ds(0, FC)], sem)

    def drain(in_v, sem):
        pltpu.make_async_copy(tab_t_hbm.at[:, pl.ds(0, FC)],
                              in_v.at[:, pl.ds(0, FC)], sem).wait()

    def put(b, out_v, sem):
        pltpu.async_copy(out_v, out_hbm.at[pl.ds(FC * b, FC)], sem)

    def put_drain(out_v, sem):
        pltpu.make_async_copy(out_v, out_hbm.at[pl.ds(0, FC)], sem).wait()

    fire(wid, in0, sem0)

    def pair_body(t, _):
        b0 = wid + NW * 2 * t

        @pl.when(b0 + NW < NBLK)
        def _():
            fire(b0 + NW, in1, sem1)

        @pl.when(b0 < NBLK)
        def _():
            drain(in0, sem0)

            @pl.when(t > 0)
            def _():
                put_drain(out0, wsem0)
            transpose_block(in0, out0)
            put(b0, out0, wsem0)

        @pl.when(b0 + 2 * NW < NBLK)
        def _():
            fire(b0 + 2 * NW, in0, sem0)

        @pl.when(b0 + NW < NBLK)
        def _():
            drain(in1, sem1)

            @pl.when(t > 0)
            def _():
                put_drain(out1, wsem1)
            transpose_block(in1, out1)
            put(b0 + NW, out1, wsem1)
        return 0

    lax.fori_loop(0, (BLK_PER_W + 1) // 2, pair_body, 0)
    put_drain(out0, wsem0)
    put_drain(out1, wsem1)

    @pl.when(wid == 0)
    def _():
        pltpu.sync_copy(tail_hbm, tail_in)

        def tail_body(r, _):
            for k in range(4):
                tail_out[r, pl.ds(16 * k, 16)] = tail_in[r, pl.ds(16 * k, 16)]
            return 0
        lax.fori_loop(0, TAIL, tail_body, 0)
        pltpu.sync_copy(tail_out, out_hbm.at[pl.ds(BULK, TAIL)])


_fmt_call = functools.partial(
    pl.kernel,
    out_type=jax.ShapeDtypeStruct((VOCAB, 2 * EMB), jnp.float32),
    mesh=plsc.VectorSubcoreMesh(core_axis_name="c", subcore_axis_name="s"),
    compiler_params=pltpu.CompilerParams(needs_layout_passes=False,
                                         disable_bounds_checks=True),
    scratch_types=[
        pltpu.VMEM((EMB, FC + 1), jnp.float32),
        pltpu.VMEM((EMB, FC + 1), jnp.float32),
        pltpu.VMEM((FC, 2 * EMB), jnp.float32),
        pltpu.VMEM((FC, 2 * EMB), jnp.float32),
        pltpu.VMEM((TAIL, EMB), jnp.float32),
        pltpu.VMEM((TAIL, 2 * EMB), jnp.float32),
        pltpu.SemaphoreType.DMA,
        pltpu.SemaphoreType.DMA,
        pltpu.SemaphoreType.DMA,
        pltpu.SemaphoreType.DMA,
    ],
)(_fmt_body)


ROW_W = EMB


def _pool_body(idx_hbm, table_hbm, out_hbm, idx_v, buf0, buf1, stage,
               sem0, sem1):
    wid = lax.axis_index("s") * NC + lax.axis_index("c")
    base = wid * B_PER_W

    pltpu.sync_copy(idx_hbm.at[pl.ds(wid * TOK_PER_W, TOK_PER_W)], idx_v)

    def fire(i, buf, sem):
        tok = i * MAX_LEN
        pltpu.async_copy(table_hbm.at[idx_v.at[pl.ds(tok, S0)]],
                         buf.at[pl.ds(0, S0)], sem)
        pltpu.async_copy(table_hbm.at[idx_v.at[pl.ds(tok + S0, S1)]],
                         buf.at[pl.ds(S0, S1)], sem)

    def drain(buf, sem):
        pltpu.make_async_copy(table_hbm.at[idx_v.at[pl.ds(0, S0)]],
                              buf.at[pl.ds(0, S0)], sem).wait()
        pltpu.make_async_copy(table_hbm.at[idx_v.at[pl.ds(0, S1)]],
                              buf.at[pl.ds(S0, S1)], sem).wait()

    zero = jnp.zeros((16,), jnp.float32)

    def reduce_item(i, buf):
        def red(m, accs):
            a = tuple(
                accs[k] + buf[2 * m, pl.ds(16 * k, 16)] for k in range(4)
            )
            b = tuple(
                accs[4 + k] + buf[2 * m + 1, pl.ds(16 * k, 16)]
                for k in range(4)
            )
            return a + b
        accs = lax.fori_loop(0, MAX_LEN // 2, red, (zero,) * 8)
        for k in range(4):
            stage[i, pl.ds(16 * k, 16)] = accs[k] + accs[4 + k]

    fire(0, buf0, sem0)

    def pair_body(g, _):
        i0 = 2 * g
        fire(i0 + 1, buf1, sem1)
        drain(buf0, sem0)
        reduce_item(i0, buf0)

        @pl.when(g < B_PER_W // 2 - 1)
        def _():
            fire(i0 + 2, buf0, sem0)
        drain(buf1, sem1)
        reduce_item(i0 + 1, buf1)
        return 0

    lax.fori_loop(0, B_PER_W // 2, pair_body, 0)
    pltpu.sync_copy(stage, out_hbm.at[pl.ds(base, B_PER_W)])


_pool_call = functools.partial(
    pl.kernel,
    out_type=jax.ShapeDtypeStruct((BATCH, EMB), jnp.float32),
    mesh=plsc.VectorSubcoreMesh(core_axis_name="c", subcore_axis_name="s"),
    compiler_params=pltpu.CompilerParams(use_tc_tiling_on_sc=False),
    scratch_types=[
        pltpu.VMEM((TOK_PER_W,), jnp.int32),
        pltpu.VMEM((MAX_LEN, ROW_W), jnp.float32),
        pltpu.VMEM((MAX_LEN, ROW_W), jnp.float32),
        pltpu.VMEM((B_PER_W, EMB), jnp.float32),
        pltpu.SemaphoreType.DMA,
        pltpu.SemaphoreType.DMA,
    ],
)(_pool_body)


CPAD = 128
_DBLK = 512


def _dense_kernel(x_ref, w_ref, b_ref, o_ref):
    logits = jnp.dot(x_ref[...], w_ref[...],
                     preferred_element_type=jnp.float32) + b_ref[...]
    m = jnp.max(logits, axis=-1, keepdims=True)
    e = jnp.exp(logits - m)
    o_ref[...] = e / jnp.sum(e, axis=-1, keepdims=True)


_dense_call = pl.pallas_call(
    _dense_kernel,
    grid=(BATCH // _DBLK,),
    in_specs=[
        pl.BlockSpec((_DBLK, EMB), lambda i: (i, 0)),
        pl.BlockSpec((EMB, CPAD), lambda i: (0, 0)),
        pl.BlockSpec((1, CPAD), lambda i: (0, 0)),
    ],
    out_specs=pl.BlockSpec((_DBLK, CPAD), lambda i: (i, 0)),
    out_shape=jax.ShapeDtypeStruct((BATCH, CPAD), jnp.float32),
)


def kernel(inputs, table, W, b):
    idx = inputs.astype(jnp.int32).reshape(-1)
    pool_sum = _pool_call(idx, table)
    w_pad = jnp.pad(W * (1.0 / MAX_LEN), ((0, 0), (0, CPAD - CLASSES)))
    b_pad = jnp.concatenate(
        [b, jnp.full((CPAD - CLASSES,), -1e30, b.dtype)]).reshape(1, CPAD)
    out = _dense_call(pool_sum, w_pad, b_pad)
    return out[:, :CLASSES]
